# Initial kernel scaffold; baseline (speedup 1.0000x reference)
import jax
import jax.numpy as jnp
from jax.experimental import pallas as pl


def kernel(x, embed, adj, W_in, b_in, g_in, be_in, W_emb, b_emb, g_emb, be_emb, W_s0, W_s1, Wih0, Whh0, bih0, bhh0, Wih1, Whh1, bih1, bhh1, W_fc, b_fc, g_fc, be_fc, W_out, b_out):
    raise NotImplementedError("write your pallas kernel here")


# speedup vs baseline: 3.6186x; 3.6186x over previous
import functools

import jax
import jax.numpy as jnp
from jax import lax
from jax.experimental import pallas as pl
from jax.experimental.pallas import tpu as pltpu
from jax.experimental.pallas import tpu_sc as plsc


def _elu(x):
    return jnp.where(x > 0, x, jnp.exp(jnp.minimum(x, 0.0)) - 1.0)


def _sigmoid(x):
    return 1.0 / (1.0 + jnp.exp(-x))


def _bn(h, g, b):
    m = jnp.mean(h, axis=0, keepdims=True)
    v = jnp.mean((h - m) ** 2, axis=0, keepdims=True)
    return (h - m) / jnp.sqrt(v + 1e-5) * g + b


def _mm_t(a, w):
    return lax.dot_general(a, w, (((1,), (1,)), ((), ())),
                           preferred_element_type=jnp.float32)


def _split2(h):
    half = h.shape[1] // 2
    return jnp.stack([h[:, :half], h[:, half:]], axis=0)


_CH = 128


@functools.lru_cache(maxsize=None)
def _build_agg(n_nodes, half, n_chunks_t, with_deg):
    stripe = -(-(n_nodes + 1) // (16 * 8)) * 8
    np_rows = 16 * stripe
    assert np_rows > n_nodes
    out_tiles = next(t for t in range(16, 0, -1)
                     if n_nodes % t == 0 and (n_nodes // t) % 8 == 0)
    rows_out = n_nodes // out_tiles

    out_type = [jax.ShapeDtypeStruct((2, n_nodes, half), jnp.float32)]
    if with_deg:
        out_type.append(jax.ShapeDtypeStruct((n_nodes, 16), jnp.float32))

    scratch = [
        pltpu.VMEM((n_chunks_t, _CH), jnp.int32),
        pltpu.VMEM((n_chunks_t, _CH), jnp.int32),
        pltpu.VMEM((_CH, half), jnp.float32),
        pltpu.VMEM((_CH, 16), jnp.float32),
        pltpu.SemaphoreType.DMA,
        pltpu.VMEM_SHARED((np_rows, half), jnp.float32),
        pltpu.VMEM_SHARED((np_rows, 16), jnp.float32),
    ]

    mesh = plsc.VectorSubcoreMesh(core_axis_name="c", subcore_axis_name="s")

    @functools.partial(
        pl.kernel, out_type=out_type, mesh=mesh, scratch_types=scratch,
        compiler_params=pltpu.CompilerParams(use_tc_tiling_on_sc=False))
    def agg(h_hbm, src_hbm, dst_hbm, zrow_hbm, zdeg_hbm, ones_hbm,
            out_sum, *rest):
        if with_deg:
            out_deg = rest[0]
            rest = rest[1:]
        src_v, dst_v, rows_v, ones_v, sem, acc_sh, deg_sh = rest

        c = lax.axis_index("c")
        s = lax.axis_index("s")

        pltpu.sync_copy(zrow_hbm.at[pl.ds(s * stripe, stripe)],
                        acc_sh.at[pl.ds(s * stripe, stripe)])
        if with_deg:
            @pl.when(c == 0)
            def _():
                pltpu.sync_copy(zdeg_hbm.at[pl.ds(s * stripe, stripe)],
                                deg_sh.at[pl.ds(s * stripe, stripe)])
                pltpu.sync_copy(ones_hbm, ones_v)

        pltpu.sync_copy(src_hbm.at[pl.ds(s * n_chunks_t, n_chunks_t)], src_v)
        pltpu.sync_copy(dst_hbm.at[pl.ds(s * n_chunks_t, n_chunks_t)], dst_v)
        plsc.subcore_barrier()

        def chunk(j, carry):
            pltpu.async_copy(h_hbm.at[c].at[src_v.at[j]], rows_v, sem).wait()
            pltpu.sync_copy(rows_v, acc_sh.at[dst_v.at[j]], add=True)
            if with_deg:
                @pl.when(c == 0)
                def _():
                    pltpu.sync_copy(ones_v, deg_sh.at[dst_v.at[j]], add=True)
            return carry

        lax.fori_loop(0, n_chunks_t, chunk, 0)
        plsc.subcore_barrier()

        @pl.when(s < out_tiles)
        def _():
            pltpu.sync_copy(acc_sh.at[pl.ds(s * rows_out, rows_out)],
                            out_sum.at[c, pl.ds(s * rows_out, rows_out)])
            if with_deg:
                @pl.when(c == 0)
                def _():
                    pltpu.sync_copy(
                        deg_sh.at[pl.ds(s * rows_out, rows_out)],
                        out_deg.at[pl.ds(s * rows_out, rows_out)])

    return agg


def _aggregate(hsplit, src2d, dst2d, zrow, zdeg, ones, with_deg):
    _, n_nodes, half = hsplit.shape
    n_chunks_t = src2d.shape[0] // 16
    agg = _build_agg(n_nodes, half, n_chunks_t, with_deg)
    return agg(hsplit, src2d, dst2d, zrow, zdeg, ones)


def _in_body(x_ref, w_ref, b_ref, g_ref, be_ref, o_ref):
    h = _mm_t(x_ref[...], w_ref[...]) + b_ref[...]
    o_ref[...] = _split2(_elu(_bn(h, g_ref[...], be_ref[...])))


def _sage_body(hs_ref, ss_ref, dg_ref, w_ref, o_ref):
    h = jnp.concatenate([hs_ref[0], hs_ref[1]], axis=1)
    summed = jnp.concatenate([ss_ref[0], ss_ref[1]], axis=1)
    neigh = summed / jnp.maximum(dg_ref[:, 0:1], 1.0)
    combined = jnp.concatenate([h, neigh], axis=1)
    out = jnp.maximum(_mm_t(combined, w_ref[...]), 0.0)
    nrm = jnp.sqrt(jnp.sum(out * out, axis=1, keepdims=True))
    o_ref[...] = _split2(out / jnp.maximum(nrm, 1e-12))


def _lstm_layer(x0, x1, wih, whh, bih, bhh):
    nh = whh.shape[1]

    def step(h, c, xt):
        gates = _mm_t(xt, wih) + bih + _mm_t(h, whh) + bhh
        i = _sigmoid(gates[:, 0 * nh:1 * nh])
        f = _sigmoid(gates[:, 1 * nh:2 * nh])
        g = jnp.tanh(gates[:, 2 * nh:3 * nh])
        o = _sigmoid(gates[:, 3 * nh:4 * nh])
        c = f * c + i * g
        h = o * jnp.tanh(c)
        return h, c

    z = jnp.zeros_like(x0)
    h0, c0 = step(z, z, x0)
    h1, _ = step(h0, c0, x1)
    return h0, h1


def _head_body(h1_ref, h2_ref, e_ref,
               wih0_ref, whh0_ref, bih0_ref, bhh0_ref,
               wih1_ref, whh1_ref, bih1_ref, bhh1_ref,
               g_in_ref, be_in_ref,
               we_ref, be_ref, g_e_ref, bee_ref,
               wfc_ref, bfc_ref, gfc_ref, befc_ref,
               wout_ref, bout_ref, o_ref):
    h1 = jnp.concatenate([h1_ref[0], h1_ref[1]], axis=1)
    h2 = jnp.concatenate([h2_ref[0], h2_ref[1]], axis=1)
    a0, a1 = _lstm_layer(h1, h2,
                         wih0_ref[...], whh0_ref[...],
                         bih0_ref[...], bhh0_ref[...])
    b0, b1 = _lstm_layer(a0, a1,
                         wih1_ref[...], whh1_ref[...],
                         bih1_ref[...], bhh1_ref[...])
    h = 0.5 * (b0 + b1)
    h = _elu(_bn(h, g_in_ref[...], be_in_ref[...]))
    e = _mm_t(e_ref[...], we_ref[...]) + be_ref[...]
    e = _elu(_bn(e, g_e_ref[...], bee_ref[...]))
    he = jnp.concatenate([h, e], axis=1)
    f = _mm_t(he, wfc_ref[...]) + bfc_ref[...]
    f = _elu(_bn(f, gfc_ref[...], befc_ref[...]))
    y = _mm_t(f, wout_ref[...]) + bout_ref[...]
    ymax = jnp.max(y, axis=1, keepdims=True)
    sh = y - ymax
    o_ref[...] = sh - jnp.log(jnp.sum(jnp.exp(sh), axis=1, keepdims=True))


def _tc(body, out_shape, *args):
    params = pltpu.CompilerParams(vmem_limit_bytes=100 * 1024 * 1024)
    return pl.pallas_call(body, out_shape=out_shape,
                          compiler_params=params)(*args)


def kernel(x, embed, adj, W_in, b_in, g_in, be_in, W_emb, b_emb, g_emb,
           be_emb, W_s0, W_s1, Wih0, Whh0, bih0, bhh0, Wih1, Whh1, bih1,
           bhh1, W_fc, b_fc, g_fc, be_fc, W_out, b_out):
    n = x.shape[0]
    nhid = W_in.shape[0]
    half = nhid // 2
    f32 = jnp.float32

    src = adj[0].astype(jnp.int32)
    dst = adj[1].astype(jnp.int32)
    e = src.shape[0]
    n_ch = -(-e // _CH)
    n_ch_pad = -(-n_ch // (16 * 8)) * (16 * 8)
    e_pad = n_ch_pad * _CH
    pad = e_pad - e
    src2d = jnp.concatenate([src, jnp.zeros((pad,), jnp.int32)]).reshape(
        n_ch_pad, _CH)
    dst2d = jnp.concatenate([dst, jnp.full((pad,), n, jnp.int32)]).reshape(
        n_ch_pad, _CH)

    stripe = -(-(n + 1) // (16 * 8)) * 8
    np_rows = 16 * stripe
    zrow = jnp.zeros((np_rows, half), f32)
    zdeg = jnp.zeros((np_rows, 16), f32)
    ones = jnp.ones((_CH, 16), f32)

    row = lambda v: v.reshape(1, -1)

    h0s = _tc(_in_body, jax.ShapeDtypeStruct((2, n, half), f32),
              x, W_in, row(b_in), row(g_in), row(be_in))

    s0, deg = _aggregate(h0s, src2d, dst2d, zrow, zdeg, ones, True)
    h1s = _tc(_sage_body, jax.ShapeDtypeStruct((2, n, half), f32),
              h0s, s0, deg, W_s0)

    (s1,) = _aggregate(h1s, src2d, dst2d, zrow, zdeg, ones, False)
    h2s = _tc(_sage_body, jax.ShapeDtypeStruct((2, n, half), f32),
              h1s, s1, deg, W_s1)

    nout = W_out.shape[0]
    out = _tc(_head_body, jax.ShapeDtypeStruct((n, nout), f32),
              h1s, h2s, embed,
              Wih0, Whh0, row(bih0), row(bhh0),
              Wih1, Whh1, row(bih1), row(bhh1),
              row(g_in), row(be_in),
              W_emb, row(b_emb), row(g_emb), row(be_emb),
              W_fc, row(b_fc), row(g_fc), row(be_fc),
              W_out, row(b_out))
    return out


# speedup vs baseline: 4.7938x; 1.3248x over previous
import functools

import jax
import jax.numpy as jnp
from jax import lax
from jax.experimental import pallas as pl
from jax.experimental.pallas import tpu as pltpu
from jax.experimental.pallas import tpu_sc as plsc


def _elu(x):
    return jnp.where(x > 0, x, jnp.exp(jnp.minimum(x, 0.0)) - 1.0)


def _sigmoid(x):
    return 1.0 / (1.0 + jnp.exp(-x))


def _bn(h, g, b):
    m = jnp.mean(h, axis=0, keepdims=True)
    v = jnp.mean((h - m) ** 2, axis=0, keepdims=True)
    return (h - m) / jnp.sqrt(v + 1e-5) * g + b


def _mm_t(a, w):
    return lax.dot_general(a, w, (((1,), (1,)), ((), ())),
                           preferred_element_type=jnp.float32)


def _split2(h):
    half = h.shape[1] // 2
    return jnp.stack([h[:, :half], h[:, half:]], axis=0)


_CH = 128


@functools.lru_cache(maxsize=None)
def _build_agg(n_nodes, half, n_chunks_t, with_deg):
    stripe = -(-(n_nodes + 1) // (16 * 8)) * 8
    np_rows = 16 * stripe
    assert np_rows > n_nodes
    out_tiles = next(t for t in range(16, 0, -1)
                     if n_nodes % t == 0 and (n_nodes // t) % 8 == 0)
    rows_out = n_nodes // out_tiles

    out_type = [jax.ShapeDtypeStruct((2, n_nodes, half), jnp.float32)]
    if with_deg:
        out_type.append(jax.ShapeDtypeStruct((n_nodes, 16), jnp.float32))

    nbuf = 4
    assert n_chunks_t % nbuf == 0
    scratch = [
        pltpu.VMEM((n_chunks_t, _CH), jnp.int32),
        pltpu.VMEM((n_chunks_t, _CH), jnp.int32),
        pltpu.VMEM((nbuf, _CH, half), jnp.float32),
        pltpu.VMEM((_CH, 16), jnp.float32),
        pltpu.SemaphoreType.DMA((nbuf,)),
        pltpu.SemaphoreType.DMA((nbuf,)),
        pltpu.SemaphoreType.DMA,
        pltpu.VMEM_SHARED((np_rows, half), jnp.float32),
        pltpu.VMEM_SHARED((np_rows, 16), jnp.float32),
    ]

    mesh = plsc.VectorSubcoreMesh(core_axis_name="c", subcore_axis_name="s")

    @functools.partial(
        pl.kernel, out_type=out_type, mesh=mesh, scratch_types=scratch,
        compiler_params=pltpu.CompilerParams(use_tc_tiling_on_sc=False))
    def agg(h_hbm, src_hbm, dst_hbm, zrow_hbm, zdeg_hbm, ones_hbm,
            out_sum, *rest):
        if with_deg:
            out_deg = rest[0]
            rest = rest[1:]
        (src_v, dst_v, rows_v, ones_v, sem_g, sem_s, sem_d,
         acc_sh, deg_sh) = rest

        c = lax.axis_index("c")
        s = lax.axis_index("s")

        pltpu.sync_copy(zrow_hbm.at[pl.ds(s * stripe, stripe)],
                        acc_sh.at[pl.ds(s * stripe, stripe)])
        if with_deg:
            @pl.when(c == 0)
            def _():
                pltpu.sync_copy(zdeg_hbm.at[pl.ds(s * stripe, stripe)],
                                deg_sh.at[pl.ds(s * stripe, stripe)])
                pltpu.sync_copy(ones_hbm, ones_v)

        pltpu.sync_copy(src_hbm.at[pl.ds(s * n_chunks_t, n_chunks_t)], src_v)
        pltpu.sync_copy(dst_hbm.at[pl.ds(s * n_chunks_t, n_chunks_t)], dst_v)
        plsc.subcore_barrier()

        def start_gather(j, b):
            pltpu.async_copy(h_hbm.at[c].at[src_v.at[j]], rows_v.at[b],
                             sem_g.at[b])

        for b in range(nbuf):
            start_gather(b, b)

        def group(g, carry):
            for b in range(nbuf):
                j = g * nbuf + b
                pltpu.make_async_copy(h_hbm.at[c].at[src_v.at[j]],
                                      rows_v.at[b], sem_g.at[b]).wait()
                pltpu.async_copy(rows_v.at[b], acc_sh.at[dst_v.at[j]],
                                 sem_s.at[b], add=True)
                if with_deg:
                    @pl.when(c == 0)
                    def _():
                        pltpu.async_copy(ones_v, deg_sh.at[dst_v.at[j]],
                                         sem_d, add=True)
                jn = j + nbuf

                @pl.when(jn < n_chunks_t)
                def _():
                    pltpu.make_async_copy(rows_v.at[b],
                                          acc_sh.at[dst_v.at[j]],
                                          sem_s.at[b]).wait()
                    start_gather(jn, b)
            if with_deg:
                @pl.when(c == 0)
                def _():
                    for b in range(nbuf):
                        pltpu.make_async_copy(
                            ones_v, deg_sh.at[dst_v.at[g * nbuf + b]],
                            sem_d).wait()
            return carry

        lax.fori_loop(0, n_chunks_t // nbuf, group, 0)
        for b in range(nbuf):
            j = n_chunks_t - nbuf + b
            pltpu.make_async_copy(rows_v.at[b], acc_sh.at[dst_v.at[j]],
                                  sem_s.at[b]).wait()
        plsc.subcore_barrier()

        @pl.when(s < out_tiles)
        def _():
            pltpu.sync_copy(acc_sh.at[pl.ds(s * rows_out, rows_out)],
                            out_sum.at[c, pl.ds(s * rows_out, rows_out)])
            if with_deg:
                @pl.when(c == 0)
                def _():
                    pltpu.sync_copy(
                        deg_sh.at[pl.ds(s * rows_out, rows_out)],
                        out_deg.at[pl.ds(s * rows_out, rows_out)])

    return agg


def _aggregate(hsplit, src2d, dst2d, zrow, zdeg, ones, with_deg):
    _, n_nodes, half = hsplit.shape
    n_chunks_t = src2d.shape[0] // 16
    agg = _build_agg(n_nodes, half, n_chunks_t, with_deg)
    return agg(hsplit, src2d, dst2d, zrow, zdeg, ones)


def _in_body(x_ref, w_ref, b_ref, g_ref, be_ref, o_ref):
    h = _mm_t(x_ref[...], w_ref[...]) + b_ref[...]
    o_ref[...] = _split2(_elu(_bn(h, g_ref[...], be_ref[...])))


def _sage_body(hs_ref, ss_ref, dg_ref, w_ref, o_ref):
    h = jnp.concatenate([hs_ref[0], hs_ref[1]], axis=1)
    summed = jnp.concatenate([ss_ref[0], ss_ref[1]], axis=1)
    neigh = summed / jnp.maximum(dg_ref[:, 0:1], 1.0)
    combined = jnp.concatenate([h, neigh], axis=1)
    out = jnp.maximum(_mm_t(combined, w_ref[...]), 0.0)
    nrm = jnp.sqrt(jnp.sum(out * out, axis=1, keepdims=True))
    o_ref[...] = _split2(out / jnp.maximum(nrm, 1e-12))


def _lstm_layer(x0, x1, wih, whh, bih, bhh):
    nh = whh.shape[1]

    def step(h, c, xt):
        gates = _mm_t(xt, wih) + bih + _mm_t(h, whh) + bhh
        i = _sigmoid(gates[:, 0 * nh:1 * nh])
        f = _sigmoid(gates[:, 1 * nh:2 * nh])
        g = jnp.tanh(gates[:, 2 * nh:3 * nh])
        o = _sigmoid(gates[:, 3 * nh:4 * nh])
        c = f * c + i * g
        h = o * jnp.tanh(c)
        return h, c

    z = jnp.zeros_like(x0)
    h0, c0 = step(z, z, x0)
    h1, _ = step(h0, c0, x1)
    return h0, h1


def _head_body(h1_ref, h2_ref, e_ref,
               wih0_ref, whh0_ref, bih0_ref, bhh0_ref,
               wih1_ref, whh1_ref, bih1_ref, bhh1_ref,
               g_in_ref, be_in_ref,
               we_ref, be_ref, g_e_ref, bee_ref,
               wfc_ref, bfc_ref, gfc_ref, befc_ref,
               wout_ref, bout_ref, o_ref):
    h1 = jnp.concatenate([h1_ref[0], h1_ref[1]], axis=1)
    h2 = jnp.concatenate([h2_ref[0], h2_ref[1]], axis=1)
    a0, a1 = _lstm_layer(h1, h2,
                         wih0_ref[...], whh0_ref[...],
                         bih0_ref[...], bhh0_ref[...])
    b0, b1 = _lstm_layer(a0, a1,
                         wih1_ref[...], whh1_ref[...],
                         bih1_ref[...], bhh1_ref[...])
    h = 0.5 * (b0 + b1)
    h = _elu(_bn(h, g_in_ref[...], be_in_ref[...]))
    e = _mm_t(e_ref[...], we_ref[...]) + be_ref[...]
    e = _elu(_bn(e, g_e_ref[...], bee_ref[...]))
    he = jnp.concatenate([h, e], axis=1)
    f = _mm_t(he, wfc_ref[...]) + bfc_ref[...]
    f = _elu(_bn(f, gfc_ref[...], befc_ref[...]))
    y = _mm_t(f, wout_ref[...]) + bout_ref[...]
    ymax = jnp.max(y, axis=1, keepdims=True)
    sh = y - ymax
    o_ref[...] = sh - jnp.log(jnp.sum(jnp.exp(sh), axis=1, keepdims=True))


def _tc(body, out_shape, *args):
    params = pltpu.CompilerParams(vmem_limit_bytes=100 * 1024 * 1024)
    return pl.pallas_call(body, out_shape=out_shape,
                          compiler_params=params)(*args)


def kernel(x, embed, adj, W_in, b_in, g_in, be_in, W_emb, b_emb, g_emb,
           be_emb, W_s0, W_s1, Wih0, Whh0, bih0, bhh0, Wih1, Whh1, bih1,
           bhh1, W_fc, b_fc, g_fc, be_fc, W_out, b_out):
    n = x.shape[0]
    nhid = W_in.shape[0]
    half = nhid // 2
    f32 = jnp.float32

    src = adj[0].astype(jnp.int32)
    dst = adj[1].astype(jnp.int32)
    e = src.shape[0]
    n_ch = -(-e // _CH)
    n_ch_pad = -(-n_ch // (16 * 8)) * (16 * 8)
    e_pad = n_ch_pad * _CH
    pad = e_pad - e
    src2d = jnp.concatenate([src, jnp.zeros((pad,), jnp.int32)]).reshape(
        n_ch_pad, _CH)
    dst2d = jnp.concatenate([dst, jnp.full((pad,), n, jnp.int32)]).reshape(
        n_ch_pad, _CH)

    stripe = -(-(n + 1) // (16 * 8)) * 8
    np_rows = 16 * stripe
    zrow = jnp.zeros((np_rows, half), f32)
    zdeg = jnp.zeros((np_rows, 16), f32)
    ones = jnp.ones((_CH, 16), f32)

    row = lambda v: v.reshape(1, -1)

    h0s = _tc(_in_body, jax.ShapeDtypeStruct((2, n, half), f32),
              x, W_in, row(b_in), row(g_in), row(be_in))

    s0, deg = _aggregate(h0s, src2d, dst2d, zrow, zdeg, ones, True)
    h1s = _tc(_sage_body, jax.ShapeDtypeStruct((2, n, half), f32),
              h0s, s0, deg, W_s0)

    (s1,) = _aggregate(h1s, src2d, dst2d, zrow, zdeg, ones, False)
    h2s = _tc(_sage_body, jax.ShapeDtypeStruct((2, n, half), f32),
              h1s, s1, deg, W_s1)

    nout = W_out.shape[0]
    out = _tc(_head_body, jax.ShapeDtypeStruct((n, nout), f32),
              h1s, h2s, embed,
              Wih0, Whh0, row(bih0), row(bhh0),
              Wih1, Whh1, row(bih1), row(bhh1),
              row(g_in), row(be_in),
              W_emb, row(b_emb), row(g_emb), row(be_emb),
              W_fc, row(b_fc), row(g_fc), row(be_fc),
              W_out, row(b_out))
    return out
